# baseline (speedup 1.0000x reference)
import functools

import numpy as np
import jax
import jax.numpy as jnp
from jax import lax
from jax.experimental import pallas as pl
from jax.experimental.pallas import tpu as pltpu
from jax.experimental.pallas import tpu_sc as plsc

FDIMS = [20, 38, 119, 4, 12, 12, 10, 6, 6, 2, 2, 4]
NC = len(FDIMS)
OFFS = np.concatenate([[0], np.cumsum(FDIMS)]).astype(np.int32)
VOCAB = int(OFFS[-1])
VPAD = 256
EMB = 128
SIGMA = 32
PEP = 16
BLK = 8192
BLKF = 5000

GROUPS = [(2, 9, 10), (0, 7, 8), (1, 3, 11), (4, 5, 6)]
GSIZES = [int(np.prod([FDIMS[f] for f in g])) for g in GROUPS]
GBASES = np.concatenate([[0], np.cumsum(GSIZES)]).astype(np.int32)
RJT = int(GBASES[-1])
RJT_PAD = (RJT + 7) // 8 * 8

NWORK = 32
CHUNK = 80
NHALF = 50000
NCHUNK_HALF = 625
TRIPS = 20
PKW = EMB // 2


def _expansion_matrix():
    k = np.zeros((RJT_PAD, VPAD), np.float32)
    for g, feats in enumerate(GROUPS):
        dims = [FDIMS[f] for f in feats]
        base = int(GBASES[g])
        for r in range(GSIZES[g]):
            rr, idxs = r, []
            for d in reversed(dims):
                idxs.append(rr % d)
                rr //= d
            idxs.reverse()
            for f, i in zip(feats, idxs):
                k[base + r, int(OFFS[f]) + i] = 1.0
    return jnp.asarray(k, dtype=jnp.bfloat16)


def _prep_body(t_ref, wlin_ref, wfin_ref, blin_ref, bfin_ref, k_ref,
               jt_ref, w48_ref, bp_ref):
    wf1 = wfin_ref[0:EMB, :]
    tp = jnp.dot(t_ref[...], wf1, preferred_element_type=jnp.float32)
    jt_ref[...] = jnp.dot(k_ref[...], tp.astype(jnp.bfloat16),
                          preferred_element_type=jnp.float32)
    w48_ref[0:SIGMA, :] = jnp.dot(wlin_ref[...], wf1,
                                  preferred_element_type=jnp.float32
                                  ).astype(jnp.bfloat16)
    w48_ref[SIGMA:SIGMA + PEP, :] = wfin_ref[EMB:EMB + PEP, :].astype(
        jnp.bfloat16)
    bp_ref[...] = (jnp.dot(blin_ref[...], wf1,
                           preferred_element_type=jnp.float32)
                   + bfin_ref[...])


def _jidx_coef():
    mct = np.zeros((8, 16), np.float32)
    for g, feats in enumerate(GROUPS):
        dims = [FDIMS[f] for f in feats]
        coef = 1
        for j in range(len(feats) - 1, -1, -1):
            mct[g, feats[j]] = float(coef)
            coef *= dims[j]
    bases = np.zeros((8, 128), np.float32)
    for g in range(len(GROUPS)):
        bases[g, :] = float(GBASES[g])
    return (jnp.asarray(mct, dtype=jnp.bfloat16), jnp.asarray(bases))


def _jidx_body(x_ref, mct_ref, bases_ref, out_ref):
    xb = x_ref[:, 0:16].astype(jnp.bfloat16)
    jt = lax.dot_general(mct_ref[...], xb, (((1,), (1,)), ((), ())),
                         preferred_element_type=jnp.float32)
    out_ref[...] = jnp.clip((jt + bases_ref[:, 0:1]).astype(jnp.int32),
                            0, RJT - 1)


def _sc_body(row0, ntotal, jidxf_hbm, jt_hbm, g_hbm,
             j0a, j0b, j0c, j0d, j1a, j1b, j1c, j1d,
             r0a, r0b, r0c, r0d, r1a, r1b, r1c, r1d,
             semi0, semi1, semg0, semg1):
    wid = lax.axis_index("s") * 2 + lax.axis_index("c")
    nrow = ntotal
    jbufs = [[j0a, j0b, j0c, j0d], [j1a, j1b, j1c, j1d]]
    rbufs = [[r0a, r0b, r0c, r0d], [r1a, r1b, r1c, r1d]]
    semi = [semi0, semi1]
    semg = [semg0, semg1]

    def chunk_of(t):
        return wid + t * NWORK

    def fire_idx(slot, t):
        c = chunk_of(t)

        @pl.when(c < NCHUNK_HALF)
        def _():
            base = row0 + c * CHUNK
            for g in range(4):
                pltpu.async_copy(
                    jidxf_hbm.at[pl.ds(g * nrow + base, CHUNK)],
                    jbufs[slot][g], semi[slot])

    def fire_gather(slot, t):
        c = chunk_of(t)

        @pl.when(c < NCHUNK_HALF)
        def _():
            for g in range(4):
                pltpu.make_async_copy(
                    jidxf_hbm.at[pl.ds(0, CHUNK)],
                    jbufs[slot][g], semi[slot]).wait()
            for g in range(4):
                pltpu.async_copy(jt_hbm.at[jbufs[slot][g]],
                                 rbufs[slot][g], semg[slot])

    def drain(slot, t):
        c = chunk_of(t)

        @pl.when(c < NCHUNK_HALF)
        def _():
            base = c * CHUNK
            ra, rb, rc, rd = rbufs[slot]
            for g in range(4):
                pltpu.make_async_copy(jt_hbm.at[pl.ds(0, CHUNK), :],
                                      rbufs[slot][g], semg[slot]).wait()

            def acc_row(i, cy):
                for v in range(EMB // 16):
                    vs = pl.ds(v * 16, 16)
                    ra[i, vs] = ((ra[i, vs] + rb[i, vs])
                                 + (rc[i, vs] + rd[i, vs]))
                return cy

            lax.fori_loop(0, CHUNK, acc_row, 0)
            pltpu.sync_copy(ra, g_hbm.at[pl.ds(base, CHUNK), :])

    fire_idx(0, 0)
    fire_idx(1, 1)
    fire_gather(0, 0)

    def pair_body(tp, carry):
        t0 = 2 * tp
        t1 = t0 + 1
        fire_gather(1, t1)
        drain(0, t0)
        fire_idx(0, t0 + 2)
        fire_gather(0, t0 + 2)
        drain(1, t1)
        fire_idx(1, t1 + 2)
        return carry

    lax.fori_loop(0, TRIPS // 2, pair_body, 0)


def _final_body(x_ref, g_ref, w48_ref, bp_ref, out_ref):
    xb = x_ref[...].astype(jnp.bfloat16)
    acc = jnp.dot(xb[:, NC:NC + SIGMA + PEP], w48_ref[...],
                  preferred_element_type=jnp.float32)
    out_ref[...] = acc + g_ref[...] + bp_ref[0:1, :]


def _final_body2(x_ref, g_ref, w48_ref, bp_ref, prev_ref, out_ref):
    del prev_ref
    xb = x_ref[...].astype(jnp.bfloat16)
    acc = jnp.dot(xb[:, NC:NC + SIGMA + PEP], w48_ref[...],
                  preferred_element_type=jnp.float32)
    out_ref[...] = acc + g_ref[...] + bp_ref[0:1, :]


@functools.partial(jax.jit, static_argnums=())
def kernel(x, emb_0, emb_1, emb_2, emb_3, emb_4, emb_5, emb_6, emb_7, emb_8,
           emb_9, emb_10, emb_11, W_lin, b_lin, W_fin, b_fin):
    n = x.shape[0]
    tables = [emb_0, emb_1, emb_2, emb_3, emb_4, emb_5, emb_6, emb_7, emb_8,
              emb_9, emb_10, emb_11]
    t = jnp.concatenate(tables, axis=0)
    t = jnp.pad(t, ((0, VPAD - VOCAB), (0, 0)))
    blin8 = jnp.broadcast_to(b_lin[None, :], (8, EMB))
    bfin8 = jnp.broadcast_to(b_fin[None, :], (8, EMB))

    jt, w48, bp = pl.pallas_call(
        _prep_body,
        out_shape=(
            jax.ShapeDtypeStruct((RJT_PAD, EMB), jnp.float32),
            jax.ShapeDtypeStruct((SIGMA + PEP, EMB), jnp.bfloat16),
            jax.ShapeDtypeStruct((8, EMB), jnp.float32),
        ),
    )(t, W_lin, W_fin, blin8, bfin8, _expansion_matrix())

    mct, bases = _jidx_coef()
    jidx = pl.pallas_call(
        _jidx_body,
        grid=(pl.cdiv(n, BLK),),
        in_specs=[
            pl.BlockSpec((BLK, x.shape[1]), lambda i: (i, 0)),
            pl.BlockSpec((8, 16), lambda i: (0, 0)),
            pl.BlockSpec((8, EMB), lambda i: (0, 0)),
        ],
        out_specs=pl.BlockSpec((8, BLK), lambda i: (0, i)),
        out_shape=jax.ShapeDtypeStruct((8, n), jnp.int32),
        compiler_params=pltpu.CompilerParams(
            dimension_semantics=("parallel",)),
    )(x, mct, bases)

    mesh = plsc.VectorSubcoreMesh(core_axis_name="c", subcore_axis_name="s")
    jidxf = jidx.reshape(-1)
    sc_scratch = (
        [pltpu.VMEM((CHUNK,), jnp.int32)] * 8
        + [pltpu.VMEM((CHUNK, EMB), jnp.float32)] * 8
        + [pltpu.SemaphoreType.DMA] * 4
    )
    g1 = pl.kernel(
        functools.partial(_sc_body, 0, n),
        out_type=jax.ShapeDtypeStruct((NHALF, EMB), jnp.float32),
        mesh=mesh,
        scratch_types=sc_scratch,
    )(jidxf, jt)
    g2 = pl.kernel(
        functools.partial(_sc_body, NHALF, n),
        out_type=jax.ShapeDtypeStruct((NHALF, EMB), jnp.float32),
        mesh=mesh,
        scratch_types=sc_scratch,
    )(jidxf, jt)

    nb = NHALF // BLKF
    out1 = pl.pallas_call(
        _final_body,
        grid=(nb,),
        in_specs=[
            pl.BlockSpec((BLKF, x.shape[1]), lambda i: (i, 0)),
            pl.BlockSpec((BLKF, EMB), lambda i: (i, 0)),
            pl.BlockSpec((SIGMA + PEP, EMB), lambda i: (0, 0)),
            pl.BlockSpec((8, EMB), lambda i: (0, 0)),
        ],
        out_specs=pl.BlockSpec((BLKF, EMB), lambda i: (i, 0)),
        out_shape=jax.ShapeDtypeStruct((n, EMB), jnp.float32),
        compiler_params=pltpu.CompilerParams(
            dimension_semantics=("arbitrary",)),
    )(x, g1, w48, bp)
    out = pl.pallas_call(
        _final_body2,
        grid=(nb,),
        in_specs=[
            pl.BlockSpec((BLKF, x.shape[1]), lambda i: (i + nb, 0)),
            pl.BlockSpec((BLKF, EMB), lambda i: (i, 0)),
            pl.BlockSpec((SIGMA + PEP, EMB), lambda i: (0, 0)),
            pl.BlockSpec((8, EMB), lambda i: (0, 0)),
            pl.BlockSpec((8, EMB), lambda i: (0, 0)),
        ],
        out_specs=pl.BlockSpec((BLKF, EMB), lambda i: (i + nb, 0)),
        out_shape=jax.ShapeDtypeStruct((n, EMB), jnp.float32),
        input_output_aliases={4: 0},
        compiler_params=pltpu.CompilerParams(
            dimension_semantics=("arbitrary",)),
    )(x, g2, w48, bp, out1)
    return out


# speedup vs baseline: 1.0219x; 1.0219x over previous
import functools

import numpy as np
import jax
import jax.numpy as jnp
from jax import lax
from jax.experimental import pallas as pl
from jax.experimental.pallas import tpu as pltpu
from jax.experimental.pallas import tpu_sc as plsc

FDIMS = [20, 38, 119, 4, 12, 12, 10, 6, 6, 2, 2, 4]
NC = len(FDIMS)
OFFS = np.concatenate([[0], np.cumsum(FDIMS)]).astype(np.int32)
VOCAB = int(OFFS[-1])
VPAD = 256
EMB = 128
SIGMA = 32
PEP = 16
BLK = 8192
BLKF = 5000

GROUPS = [(2, 9, 10), (0, 7, 8), (1, 3, 11), (4, 5, 6)]
GSIZES = [int(np.prod([FDIMS[f] for f in g])) for g in GROUPS]
GBASES = np.concatenate([[0], np.cumsum(GSIZES)]).astype(np.int32)
RJT = int(GBASES[-1])
RJT_PAD = (RJT + 7) // 8 * 8

NWORK = 32
CHUNK = 80
NCHUNK_TOTAL = 1250
TRIPS = 40
PKW = EMB // 2


def _expansion_matrix():
    k = np.zeros((RJT_PAD, VPAD), np.float32)
    for g, feats in enumerate(GROUPS):
        dims = [FDIMS[f] for f in feats]
        base = int(GBASES[g])
        for r in range(GSIZES[g]):
            rr, idxs = r, []
            for d in reversed(dims):
                idxs.append(rr % d)
                rr //= d
            idxs.reverse()
            for f, i in zip(feats, idxs):
                k[base + r, int(OFFS[f]) + i] = 1.0
    return jnp.asarray(k, dtype=jnp.bfloat16)


def _prep_body(t_ref, wlin_ref, wfin_ref, blin_ref, bfin_ref, k_ref,
               jt_ref, w48_ref, bp_ref):
    wf1 = wfin_ref[0:EMB, :]
    tp = jnp.dot(t_ref[...], wf1, preferred_element_type=jnp.float32)
    jt_ref[...] = jnp.dot(k_ref[...], tp.astype(jnp.bfloat16),
                          preferred_element_type=jnp.float32)
    w48_ref[0:SIGMA, :] = jnp.dot(wlin_ref[...], wf1,
                                  preferred_element_type=jnp.float32
                                  ).astype(jnp.bfloat16)
    w48_ref[SIGMA:SIGMA + PEP, :] = wfin_ref[EMB:EMB + PEP, :].astype(
        jnp.bfloat16)
    bp_ref[...] = (jnp.dot(blin_ref[...], wf1,
                           preferred_element_type=jnp.float32)
                   + bfin_ref[...])


def _jidx_coef():
    mct = np.zeros((8, 16), np.float32)
    for g, feats in enumerate(GROUPS):
        dims = [FDIMS[f] for f in feats]
        coef = 1
        for j in range(len(feats) - 1, -1, -1):
            mct[g, feats[j]] = float(coef)
            coef *= dims[j]
    bases = np.zeros((8, 128), np.float32)
    for g in range(len(GROUPS)):
        bases[g, :] = float(GBASES[g])
    return (jnp.asarray(mct, dtype=jnp.bfloat16), jnp.asarray(bases))


def _jidx_body(x_ref, mct_ref, bases_ref, out_ref):
    xb = x_ref[:, 0:16].astype(jnp.bfloat16)
    jt = lax.dot_general(mct_ref[...], xb, (((1,), (1,)), ((), ())),
                         preferred_element_type=jnp.float32)
    out_ref[...] = jnp.clip((jt + bases_ref[:, 0:1]).astype(jnp.int32),
                            0, RJT - 1)


def _sc_body(row0, ntotal, jidxf_hbm, jt_hbm, g_hbm,
             j0a, j0b, j0c, j0d, j1a, j1b, j1c, j1d,
             r0a, r0b, r0c, r0d, r1a, r1b, r1c, r1d,
             semi0, semi1, semg0, semg1):
    wid = lax.axis_index("s") * 2 + lax.axis_index("c")
    nrow = ntotal
    jbufs = [[j0a, j0b, j0c, j0d], [j1a, j1b, j1c, j1d]]
    rbufs = [[r0a, r0b, r0c, r0d], [r1a, r1b, r1c, r1d]]
    semi = [semi0, semi1]
    semg = [semg0, semg1]

    def chunk_of(t):
        return wid + t * NWORK

    def fire_idx(slot, t):
        c = chunk_of(t)

        @pl.when(c < NCHUNK_TOTAL)
        def _():
            base = row0 + c * CHUNK
            for g in range(4):
                pltpu.async_copy(
                    jidxf_hbm.at[pl.ds(g * nrow + base, CHUNK)],
                    jbufs[slot][g], semi[slot])

    def fire_gather(slot, t):
        c = chunk_of(t)

        @pl.when(c < NCHUNK_TOTAL)
        def _():
            for g in range(4):
                pltpu.make_async_copy(
                    jidxf_hbm.at[pl.ds(0, CHUNK)],
                    jbufs[slot][g], semi[slot]).wait()
            for g in range(4):
                pltpu.async_copy(jt_hbm.at[jbufs[slot][g]],
                                 rbufs[slot][g], semg[slot])

    def drain(slot, t):
        c = chunk_of(t)

        @pl.when(c < NCHUNK_TOTAL)
        def _():
            base = c * CHUNK
            ra, rb, rc, rd = rbufs[slot]
            for g in range(4):
                pltpu.make_async_copy(jt_hbm.at[pl.ds(0, CHUNK), :],
                                      rbufs[slot][g], semg[slot]).wait()

            def acc_row(i, cy):
                for v in range(EMB // 16):
                    vs = pl.ds(v * 16, 16)
                    ra[i, vs] = ((ra[i, vs] + rb[i, vs])
                                 + (rc[i, vs] + rd[i, vs]))
                return cy

            lax.fori_loop(0, CHUNK, acc_row, 0)
            pltpu.sync_copy(ra, g_hbm.at[pl.ds(base, CHUNK), :])

    fire_idx(0, 0)
    fire_idx(1, 1)
    fire_gather(0, 0)

    def pair_body(tp, carry):
        t0 = 2 * tp
        t1 = t0 + 1
        fire_gather(1, t1)
        drain(0, t0)
        fire_idx(0, t0 + 2)
        fire_gather(0, t0 + 2)
        drain(1, t1)
        fire_idx(1, t1 + 2)
        return carry

    lax.fori_loop(0, TRIPS // 2, pair_body, 0)


def _final_body(x_ref, g_ref, w48_ref, bp_ref, out_ref):
    xb = x_ref[...].astype(jnp.bfloat16)
    acc = jnp.dot(xb[:, NC:NC + SIGMA + PEP], w48_ref[...],
                  preferred_element_type=jnp.float32)
    out_ref[...] = acc + g_ref[...] + bp_ref[0:1, :]


@functools.partial(jax.jit, static_argnums=())
def kernel(x, emb_0, emb_1, emb_2, emb_3, emb_4, emb_5, emb_6, emb_7, emb_8,
           emb_9, emb_10, emb_11, W_lin, b_lin, W_fin, b_fin):
    n = x.shape[0]
    tables = [emb_0, emb_1, emb_2, emb_3, emb_4, emb_5, emb_6, emb_7, emb_8,
              emb_9, emb_10, emb_11]
    t = jnp.concatenate(tables, axis=0)
    t = jnp.pad(t, ((0, VPAD - VOCAB), (0, 0)))
    blin8 = jnp.broadcast_to(b_lin[None, :], (8, EMB))
    bfin8 = jnp.broadcast_to(b_fin[None, :], (8, EMB))

    jt, w48, bp = pl.pallas_call(
        _prep_body,
        out_shape=(
            jax.ShapeDtypeStruct((RJT_PAD, EMB), jnp.float32),
            jax.ShapeDtypeStruct((SIGMA + PEP, EMB), jnp.bfloat16),
            jax.ShapeDtypeStruct((8, EMB), jnp.float32),
        ),
    )(t, W_lin, W_fin, blin8, bfin8, _expansion_matrix())

    mct, bases = _jidx_coef()
    jidx = pl.pallas_call(
        _jidx_body,
        grid=(pl.cdiv(n, BLK),),
        in_specs=[
            pl.BlockSpec((BLK, x.shape[1]), lambda i: (i, 0)),
            pl.BlockSpec((8, 16), lambda i: (0, 0)),
            pl.BlockSpec((8, EMB), lambda i: (0, 0)),
        ],
        out_specs=pl.BlockSpec((8, BLK), lambda i: (0, i)),
        out_shape=jax.ShapeDtypeStruct((8, n), jnp.int32),
        compiler_params=pltpu.CompilerParams(
            dimension_semantics=("parallel",)),
    )(x, mct, bases)

    mesh = plsc.VectorSubcoreMesh(core_axis_name="c", subcore_axis_name="s")
    jidxf = jidx.reshape(-1)
    sc_scratch = (
        [pltpu.VMEM((CHUNK,), jnp.int32)] * 8
        + [pltpu.VMEM((CHUNK, EMB), jnp.float32)] * 8
        + [pltpu.SemaphoreType.DMA] * 4
    )
    g = pl.kernel(
        functools.partial(_sc_body, 0, n),
        out_type=jax.ShapeDtypeStruct((n, EMB), jnp.float32),
        mesh=mesh,
        scratch_types=sc_scratch,
    )(jidxf, jt)

    out = pl.pallas_call(
        _final_body,
        grid=(pl.cdiv(n, BLK),),
        in_specs=[
            pl.BlockSpec((BLK, x.shape[1]), lambda i: (i, 0)),
            pl.BlockSpec((BLK, EMB), lambda i: (i, 0)),
            pl.BlockSpec((SIGMA + PEP, EMB), lambda i: (0, 0)),
            pl.BlockSpec((8, EMB), lambda i: (0, 0)),
        ],
        out_specs=pl.BlockSpec((BLK, EMB), lambda i: (i, 0)),
        out_shape=jax.ShapeDtypeStruct((n, EMB), jnp.float32),
        compiler_params=pltpu.CompilerParams(
            dimension_semantics=("parallel",)),
    )(x, g, w48, bp)
    return out


# speedup vs baseline: 1.1779x; 1.1526x over previous
import functools

import numpy as np
import jax
import jax.numpy as jnp
from jax import lax
from jax.experimental import pallas as pl
from jax.experimental.pallas import tpu as pltpu
from jax.experimental.pallas import tpu_sc as plsc

FDIMS = [20, 38, 119, 4, 12, 12, 10, 6, 6, 2, 2, 4]
NC = len(FDIMS)
OFFS = np.concatenate([[0], np.cumsum(FDIMS)]).astype(np.int32)
VOCAB = int(OFFS[-1])
VPAD = 256
EMB = 128
SIGMA = 32
PEP = 16
BLK = 8192
BLKF = 5000

GROUPS = [(2, 1), (0, 4, 5, 3), (6, 7, 8, 9, 10, 11)]
NG = len(GROUPS)
GSIZES = [int(np.prod([FDIMS[f] for f in g])) for g in GROUPS]
GBASES = np.concatenate([[0], np.cumsum(GSIZES)]).astype(np.int32)
RJT = int(GBASES[-1])
RJT_PAD = (RJT + 7) // 8 * 8

NWORK = 32
CHUNK = 160
NCHUNK_TOTAL = 625
TRIPS = 20


def _expansion_matrix():
    k = np.zeros((RJT_PAD, VPAD), np.float32)
    for g, feats in enumerate(GROUPS):
        dims = [FDIMS[f] for f in feats]
        base = int(GBASES[g])
        for r in range(GSIZES[g]):
            rr, idxs = r, []
            for d in reversed(dims):
                idxs.append(rr % d)
                rr //= d
            idxs.reverse()
            for f, i in zip(feats, idxs):
                k[base + r, int(OFFS[f]) + i] = 1.0
    return jnp.asarray(k, dtype=jnp.bfloat16)


def _prep_body(t_ref, wlin_ref, wfin_ref, blin_ref, bfin_ref, k_ref,
               jt_ref, w48_ref, bp_ref):
    wf1 = wfin_ref[0:EMB, :]
    tp = jnp.dot(t_ref[...], wf1, preferred_element_type=jnp.float32)
    jt_ref[...] = jnp.dot(k_ref[...], tp.astype(jnp.bfloat16),
                          preferred_element_type=jnp.float32)
    w48_ref[0:SIGMA, :] = jnp.dot(wlin_ref[...], wf1,
                                  preferred_element_type=jnp.float32
                                  ).astype(jnp.bfloat16)
    w48_ref[SIGMA:SIGMA + PEP, :] = wfin_ref[EMB:EMB + PEP, :].astype(
        jnp.bfloat16)
    bp_ref[...] = (jnp.dot(blin_ref[...], wf1,
                           preferred_element_type=jnp.float32)
                   + bfin_ref[...])


def _jidx_coef():
    mct = np.zeros((8, 16), np.float32)
    for g, feats in enumerate(GROUPS):
        dims = [FDIMS[f] for f in feats]
        coef = 1
        for j in range(len(feats) - 1, -1, -1):
            mct[g, feats[j]] = float(coef)
            coef *= dims[j]
    bases = np.zeros((8, 128), np.float32)
    for g in range(len(GROUPS)):
        bases[g, :] = float(GBASES[g])
    return (jnp.asarray(mct, dtype=jnp.bfloat16), jnp.asarray(bases))


def _jidx_body(x_ref, mct_ref, bases_ref, out_ref):
    xb = x_ref[:, 0:16].astype(jnp.bfloat16)
    jt = lax.dot_general(mct_ref[...], xb, (((1,), (1,)), ((), ())),
                         preferred_element_type=jnp.float32)
    out_ref[...] = jnp.clip((jt + bases_ref[:, 0:1]).astype(jnp.int32),
                            0, RJT - 1)


def _sc_body(row0, ntotal, jidxf_hbm, jt_hbm, g_hbm,
             j0a, j0b, j0c, j1a, j1b, j1c,
             r0a, r0b, r0c, r1a, r1b, r1c,
             semi0, semi1, semg0, semg1):
    wid = lax.axis_index("s") * 2 + lax.axis_index("c")
    nrow = ntotal
    jbufs = [[j0a, j0b, j0c], [j1a, j1b, j1c]]
    rbufs = [[r0a, r0b, r0c], [r1a, r1b, r1c]]
    semi = [semi0, semi1]
    semg = [semg0, semg1]

    def chunk_of(t):
        return wid + t * NWORK

    def fire_idx(slot, t):
        c = chunk_of(t)

        @pl.when(c < NCHUNK_TOTAL)
        def _():
            base = row0 + c * CHUNK
            for g in range(NG):
                pltpu.async_copy(
                    jidxf_hbm.at[pl.ds(g * nrow + base, CHUNK)],
                    jbufs[slot][g], semi[slot])

    def fire_gather(slot, t):
        c = chunk_of(t)

        @pl.when(c < NCHUNK_TOTAL)
        def _():
            for g in range(NG):
                pltpu.make_async_copy(
                    jidxf_hbm.at[pl.ds(0, CHUNK)],
                    jbufs[slot][g], semi[slot]).wait()
            for g in range(NG):
                pltpu.async_copy(jt_hbm.at[jbufs[slot][g]],
                                 rbufs[slot][g], semg[slot])

    def drain(slot, t):
        c = chunk_of(t)

        @pl.when(c < NCHUNK_TOTAL)
        def _():
            base = c * CHUNK
            ra, rb, rc = rbufs[slot]
            for g in range(NG):
                pltpu.make_async_copy(jt_hbm.at[pl.ds(0, CHUNK), :],
                                      rbufs[slot][g], semg[slot]).wait()

            def acc_row(i, cy):
                for v in range(EMB // 16):
                    vs = pl.ds(v * 16, 16)
                    ra[i, vs] = ra[i, vs] + (rb[i, vs] + rc[i, vs])
                return cy

            lax.fori_loop(0, CHUNK, acc_row, 0)
            pltpu.sync_copy(ra, g_hbm.at[pl.ds(base, CHUNK), :])

    fire_idx(0, 0)
    fire_idx(1, 1)
    fire_gather(0, 0)

    def pair_body(tp, carry):
        t0 = 2 * tp
        t1 = t0 + 1
        fire_gather(1, t1)
        drain(0, t0)
        fire_idx(0, t0 + 2)
        fire_gather(0, t0 + 2)
        drain(1, t1)
        fire_idx(1, t1 + 2)
        return carry

    lax.fori_loop(0, TRIPS // 2, pair_body, 0)


def _final_body(x_ref, g_ref, w48_ref, bp_ref, out_ref):
    xb = x_ref[...].astype(jnp.bfloat16)
    acc = jnp.dot(xb[:, NC:NC + SIGMA + PEP], w48_ref[...],
                  preferred_element_type=jnp.float32)
    out_ref[...] = acc + g_ref[...] + bp_ref[0:1, :]


@functools.partial(jax.jit, static_argnums=())
def kernel(x, emb_0, emb_1, emb_2, emb_3, emb_4, emb_5, emb_6, emb_7, emb_8,
           emb_9, emb_10, emb_11, W_lin, b_lin, W_fin, b_fin):
    n = x.shape[0]
    tables = [emb_0, emb_1, emb_2, emb_3, emb_4, emb_5, emb_6, emb_7, emb_8,
              emb_9, emb_10, emb_11]
    t = jnp.concatenate(tables, axis=0)
    t = jnp.pad(t, ((0, VPAD - VOCAB), (0, 0)))
    blin8 = jnp.broadcast_to(b_lin[None, :], (8, EMB))
    bfin8 = jnp.broadcast_to(b_fin[None, :], (8, EMB))

    jt, w48, bp = pl.pallas_call(
        _prep_body,
        out_shape=(
            jax.ShapeDtypeStruct((RJT_PAD, EMB), jnp.float32),
            jax.ShapeDtypeStruct((SIGMA + PEP, EMB), jnp.bfloat16),
            jax.ShapeDtypeStruct((8, EMB), jnp.float32),
        ),
    )(t, W_lin, W_fin, blin8, bfin8, _expansion_matrix())

    mct, bases = _jidx_coef()
    jidx = pl.pallas_call(
        _jidx_body,
        grid=(pl.cdiv(n, BLK),),
        in_specs=[
            pl.BlockSpec((BLK, x.shape[1]), lambda i: (i, 0)),
            pl.BlockSpec((8, 16), lambda i: (0, 0)),
            pl.BlockSpec((8, EMB), lambda i: (0, 0)),
        ],
        out_specs=pl.BlockSpec((8, BLK), lambda i: (0, i)),
        out_shape=jax.ShapeDtypeStruct((8, n), jnp.int32),
        compiler_params=pltpu.CompilerParams(
            dimension_semantics=("parallel",)),
    )(x, mct, bases)

    mesh = plsc.VectorSubcoreMesh(core_axis_name="c", subcore_axis_name="s")
    jidxf = jidx.reshape(-1)
    sc_scratch = (
        [pltpu.VMEM((CHUNK,), jnp.int32)] * (2 * NG)
        + [pltpu.VMEM((CHUNK, EMB), jnp.float32)] * (2 * NG)
        + [pltpu.SemaphoreType.DMA] * 4
    )
    g = pl.kernel(
        functools.partial(_sc_body, 0, n),
        out_type=jax.ShapeDtypeStruct((n, EMB), jnp.float32),
        mesh=mesh,
        scratch_types=sc_scratch,
    )(jidxf, jt)

    out = pl.pallas_call(
        _final_body,
        grid=(pl.cdiv(n, BLK),),
        in_specs=[
            pl.BlockSpec((BLK, x.shape[1]), lambda i: (i, 0)),
            pl.BlockSpec((BLK, EMB), lambda i: (i, 0)),
            pl.BlockSpec((SIGMA + PEP, EMB), lambda i: (0, 0)),
            pl.BlockSpec((8, EMB), lambda i: (0, 0)),
        ],
        out_specs=pl.BlockSpec((BLK, EMB), lambda i: (i, 0)),
        out_shape=jax.ShapeDtypeStruct((n, EMB), jnp.float32),
        compiler_params=pltpu.CompilerParams(
            dimension_semantics=("parallel",)),
    )(x, g, w48, bp)
    return out
